# Initial kernel scaffold; baseline (speedup 1.0000x reference)
import jax
import jax.numpy as jnp
from jax.experimental import pallas as pl


def kernel(users, items, user_table, bert_table, W, b):
    raise NotImplementedError("write your pallas kernel here")


# speedup vs baseline: 3.5595x; 3.5595x over previous
import jax
import jax.numpy as jnp
from jax import lax
from jax.experimental import pallas as pl
from jax.experimental.pallas import tpu as pltpu
from jax.experimental.pallas import tpu_sc as plsc

NC, NS = 2, 16
NW = NC * NS
B = 16384
B_PER_W = B // NW
CHUNK = 64
NCHUNK = B_PER_W // CHUNK
EMBED = 128
BERT = 768


def _gather_body(users_hbm, items_hbm, user_table, bert_table,
                 user_out, bert_out, uidx, iidx, ubuf, bbuf, sem):
    wid = lax.axis_index("s") * NC + lax.axis_index("c")
    base = wid * B_PER_W
    pltpu.sync_copy(users_hbm.at[wid], uidx)
    pltpu.sync_copy(items_hbm.at[wid], iidx)
    for j in range(NCHUNK):
        row0 = base + j * CHUNK
        pltpu.async_copy(user_table.at[uidx.at[j]], ubuf, sem).wait()
        pltpu.sync_copy(ubuf, user_out.at[pl.ds(row0, CHUNK)])
        pltpu.async_copy(bert_table.at[iidx.at[j]], bbuf, sem).wait()
        pltpu.sync_copy(bbuf, bert_out.at[pl.ds(row0, CHUNK)])


_gather = pl.kernel(
    _gather_body,
    out_type=(jax.ShapeDtypeStruct((B, EMBED), jnp.float32),
              jax.ShapeDtypeStruct((B, BERT), jnp.float32)),
    mesh=plsc.VectorSubcoreMesh(core_axis_name="c", subcore_axis_name="s",
                                num_cores=NC, num_subcores=NS),
    scratch_types=[
        pltpu.VMEM((NCHUNK, CHUNK), jnp.int32),
        pltpu.VMEM((NCHUNK, CHUNK), jnp.int32),
        pltpu.VMEM((CHUNK, EMBED), jnp.float32),
        pltpu.VMEM((CHUNK, BERT), jnp.float32),
        pltpu.SemaphoreType.DMA,
    ],
)

BM = 1024


def _tc_body(user_ref, bert_ref, w_ref, b_ref, out_ref):
    news = lax.dot_general(bert_ref[...], w_ref[...],
                           (((1,), (1,)), ((), ())),
                           preferred_element_type=jnp.float32)
    news = jax.nn.sigmoid(news + b_ref[...])
    out_ref[...] = jax.nn.sigmoid(jnp.sum(user_ref[...] * news, axis=1))


def kernel(users, items, user_table, bert_table, W, b):
    user_emb, bert_emb = _gather(
        users.reshape(NW, NCHUNK, CHUNK),
        items.reshape(NW, NCHUNK, CHUNK),
        user_table, bert_table)
    return pl.pallas_call(
        _tc_body,
        grid=(B // BM,),
        in_specs=[
            pl.BlockSpec((BM, EMBED), lambda i: (i, 0)),
            pl.BlockSpec((BM, BERT), lambda i: (i, 0)),
            pl.BlockSpec((EMBED, BERT), lambda i: (0, 0)),
            pl.BlockSpec((1, EMBED), lambda i: (0, 0)),
        ],
        out_specs=pl.BlockSpec((BM,), lambda i: (i,)),
        out_shape=jax.ShapeDtypeStruct((B,), jnp.float32),
    )(user_emb, bert_emb, W, b.reshape(1, EMBED))


# speedup vs baseline: 3.8549x; 1.0830x over previous
import jax
import jax.numpy as jnp
from jax import lax
from jax.experimental import pallas as pl
from jax.experimental.pallas import tpu as pltpu
from jax.experimental.pallas import tpu_sc as plsc

NC, NS = 2, 16
NW = NC * NS
B = 16384
NCHUNKS = 4
CB = B // NCHUNKS
RPW = CB // NW
SUB = 64
NSUB = RPW // SUB
EMBED = 128
BERT = 768


def _gather_body(users_hbm, items_hbm, user_table, bert_table,
                 user_out, bert_out,
                 uidx, iidx, ubuf, bb0, bb1, gs_u, ss_u, gs0, gs1, ss0, ss1):
    wid = lax.axis_index("s") * NC + lax.axis_index("c")
    base = wid * RPW
    pltpu.sync_copy(users_hbm.at[pl.ds(base, RPW)], uidx)
    pltpu.sync_copy(items_hbm.at[pl.ds(base, RPW)], iidx)
    ug = pltpu.async_copy(user_table.at[uidx], ubuf, gs_u)
    bufs, gsems, ssems = (bb0, bb1), (gs0, gs1), (ss0, ss1)
    gets = [pltpu.async_copy(bert_table.at[iidx.at[pl.ds(0, SUB)]],
                             bufs[0], gsems[0]), None]
    stores = [None, None]
    for j in range(NSUB):
        pb = j % 2
        gets[pb].wait()
        if j + 1 < NSUB:
            nb = (j + 1) % 2
            if stores[nb] is not None:
                stores[nb].wait()
                stores[nb] = None
            gets[nb] = pltpu.async_copy(
                bert_table.at[iidx.at[pl.ds((j + 1) * SUB, SUB)]],
                bufs[nb], gsems[nb])
        stores[pb] = pltpu.async_copy(
            bufs[pb], bert_out.at[pl.ds(base + j * SUB, SUB)], ssems[pb])
    ug.wait()
    us = pltpu.async_copy(ubuf, user_out.at[pl.ds(base, RPW)], ss_u)
    for st in stores:
        if st is not None:
            st.wait()
    us.wait()


_gather = pl.kernel(
    _gather_body,
    out_type=(jax.ShapeDtypeStruct((CB, EMBED), jnp.float32),
              jax.ShapeDtypeStruct((CB, BERT), jnp.float32)),
    mesh=plsc.VectorSubcoreMesh(core_axis_name="c", subcore_axis_name="s",
                                num_cores=NC, num_subcores=NS),
    scratch_types=[
        pltpu.VMEM((RPW,), jnp.int32),
        pltpu.VMEM((RPW,), jnp.int32),
        pltpu.VMEM((RPW, EMBED), jnp.float32),
        pltpu.VMEM((SUB, BERT), jnp.float32),
        pltpu.VMEM((SUB, BERT), jnp.float32),
        pltpu.SemaphoreType.DMA,
        pltpu.SemaphoreType.DMA,
        pltpu.SemaphoreType.DMA,
        pltpu.SemaphoreType.DMA,
        pltpu.SemaphoreType.DMA,
        pltpu.SemaphoreType.DMA,
    ],
)

BM = 1024


def _tc_body(user_ref, bert_ref, w_ref, b_ref, out_ref):
    news = lax.dot_general(bert_ref[...], w_ref[...],
                           (((1,), (1,)), ((), ())),
                           preferred_element_type=jnp.float32)
    news = jax.nn.sigmoid(news + b_ref[...])
    out_ref[...] = jax.nn.sigmoid(jnp.sum(user_ref[...] * news, axis=1))


_tc = pl.pallas_call(
    _tc_body,
    grid=(CB // BM,),
    in_specs=[
        pl.BlockSpec((BM, EMBED), lambda i: (i, 0)),
        pl.BlockSpec((BM, BERT), lambda i: (i, 0)),
        pl.BlockSpec((EMBED, BERT), lambda i: (0, 0)),
        pl.BlockSpec((1, EMBED), lambda i: (0, 0)),
    ],
    out_specs=pl.BlockSpec((BM,), lambda i: (i,)),
    out_shape=jax.ShapeDtypeStruct((CB,), jnp.float32),
)


def kernel(users, items, user_table, bert_table, W, b):
    b2 = b.reshape(1, EMBED)
    outs = []
    for c in range(NCHUNKS):
        ue, be = _gather(users[c * CB:(c + 1) * CB],
                         items[c * CB:(c + 1) * CB],
                         user_table, bert_table)
        outs.append(_tc(ue, be, W, b2))
    return jnp.concatenate(outs)
